# baseline (speedup 1.0000x reference)
import functools

import jax
import jax.numpy as jnp
from jax import lax
from jax.experimental import pallas as pl
from jax.experimental.pallas import tpu as pltpu
from jax.experimental.pallas import tpu_sc as plsc

Q = 1024
D = 128
C = 64
K = 10
N_BANK = 100000
QB = 128
NQB = Q // QB
BLK = 512
N_PAD = 100352
NBLK = N_PAD // BLK
RB = 16
BIGI = 2**30
NW = 32
RPW = Q * K // NW
GCH = 4
GCW = RPW // GCH


def _topk_body(x_ref, y_ref, out_idx_ref, rb_s, rb_i, m_ref, s_ref):
    pid = pl.program_id(1)

    @pl.when(pid == 0)
    def _init():
        rb_s[...] = jnp.full((RB, QB), jnp.inf, jnp.float32)
        rb_i[...] = jnp.full((RB, QB), BIGI, jnp.int32)

    x = x_ref[...]
    y = y_ref[...]
    y2 = jnp.sum(y * y, axis=1, keepdims=True)
    yx = lax.dot_general(y, x, (((1,), (1,)), ((), ())),
                         preferred_element_type=jnp.float32)
    row = lax.broadcasted_iota(jnp.int32, (BLK, QB), 0) + pid * BLK
    s = jnp.where(row < N_BANK, y2 - 2.0 * yx, jnp.inf)
    s_ref[...] = s
    m_ref[pl.ds(0, 1), :] = jnp.min(s, axis=0, keepdims=True)

    for _ in range(K):
        m = m_ref[pl.ds(0, 1), :]
        t = rb_s[pl.ds(K - 1, 1), :]

        @pl.when(jnp.any(m < t))
        def _pass():
            s = s_ref[...]
            m = m_ref[pl.ds(0, 1), :]
            ci = jnp.min(jnp.where(s == m, row, BIGI), axis=0, keepdims=True)
            t = rb_s[pl.ds(K - 1, 1), :]
            ins = m < t
            e = jnp.where(ins, m, jnp.inf)
            ie = jnp.where(ins, ci, BIGI)
            rs = rb_s[...]
            ri = rb_i[...]
            rs_sh = jnp.concatenate(
                [jnp.full((1, QB), -jnp.inf, jnp.float32), rs[:-1]], axis=0)
            ri_sh = jnp.concatenate(
                [jnp.full((1, QB), 0, jnp.int32), ri[:-1]], axis=0)
            shift = rs_sh > e
            here = rs > e
            rb_s[...] = jnp.where(shift, rs_sh, jnp.where(here, e, rs))
            rb_i[...] = jnp.where(shift, ri_sh, jnp.where(here, ie, ri))
            s2 = jnp.where(row == ci, jnp.inf, s)
            s_ref[...] = s2
            m_ref[pl.ds(0, 1), :] = jnp.min(s2, axis=0, keepdims=True)

    @pl.when(pid == NBLK - 1)
    def _emit():
        out_idx_ref[...] = rb_i[...]


def _mean_argmax_body(rows_ref, probs_ref, lab_ref):
    acc = rows_ref[0]
    for j in range(1, K):
        acc = acc + rows_ref[j]
    p = acc * jnp.float32(1.0 / K)
    probs_ref[...] = p
    m = jnp.max(p, axis=1, keepdims=True)
    ii = lax.broadcasted_iota(jnp.int32, (Q, C), 1)
    lab_ref[...] = jnp.min(jnp.where(p == m, ii, C), axis=1, keepdims=True)


@functools.cache
def _make_sc_gather():
    mesh = plsc.VectorSubcoreMesh(core_axis_name="c", subcore_axis_name="s")

    @functools.partial(
        pl.kernel,
        mesh=mesh,
        out_type=jax.ShapeDtypeStruct((Q * K, C), jnp.float32),
        scratch_types=[
            pltpu.VMEM((NW * GCH, GCW), jnp.int32),
            pltpu.VMEM((RPW, C), jnp.float32),
            pltpu.SemaphoreType.DMA,
        ],
        compiler_params=pltpu.CompilerParams(use_tc_tiling_on_sc=False),
    )
    def _sc_gather(idx_hbm, probs_hbm, out_hbm, idx_v, rows_v, sem):
        wid = lax.axis_index("s") * 2 + lax.axis_index("c")
        pltpu.sync_copy(idx_hbm, idx_v)
        copies = []
        for t in range(GCH):
            cp = pltpu.async_copy(
                probs_hbm.at[idx_v.at[wid * GCH + t]],
                rows_v.at[pl.ds(t * GCW, GCW)],
                sem,
            )
            copies.append(cp)
        for cp in copies:
            cp.wait()
        pltpu.sync_copy(rows_v, out_hbm.at[pl.ds(wid * RPW, RPW)])

    return _sc_gather


def _run_topk(features, features_bank):
    return pl.pallas_call(
        _topk_body,
        grid=(NQB, NBLK),
        in_specs=[
            pl.BlockSpec((QB, D), lambda q, i: (q, 0)),
            pl.BlockSpec((BLK, D), lambda q, i: (i, 0)),
        ],
        out_specs=pl.BlockSpec((RB, QB), lambda q, i: (0, q)),
        out_shape=jax.ShapeDtypeStruct((RB, Q), jnp.int32),
        scratch_shapes=[
            pltpu.VMEM((RB, QB), jnp.float32),
            pltpu.VMEM((RB, QB), jnp.int32),
            pltpu.VMEM((8, QB), jnp.float32),
            pltpu.VMEM((BLK, QB), jnp.float32),
        ],
    )(features, features_bank)


def _run_mean_argmax(rows3d):
    return pl.pallas_call(
        _mean_argmax_body,
        out_shape=[
            jax.ShapeDtypeStruct((Q, C), jnp.float32),
            jax.ShapeDtypeStruct((Q, 1), jnp.int32),
        ],
    )(rows3d)


def kernel(features, features_bank, probs_bank):
    top_idx = _run_topk(features, features_bank)
    idx_t = top_idx[:K].reshape(NW * GCH, GCW)
    rows = _make_sc_gather()(idx_t, probs_bank)
    probs, lab = _run_mean_argmax(rows.reshape(K, Q, C))
    return (lab[:, 0], probs)


# speedup vs baseline: 1.7113x; 1.7113x over previous
import functools

import jax
import jax.numpy as jnp
from jax import lax
from jax.experimental import pallas as pl
from jax.experimental.pallas import tpu as pltpu
from jax.experimental.pallas import tpu_sc as plsc

Q = 1024
D = 128
C = 64
K = 10
N_BANK = 100000
QB = 128
NQB = Q // QB
BLK = 512
N_PAD = 100352
NBLK = N_PAD // BLK
RB = 16
BIGI = 2**30
BIGF = 1e30
NW = 32
RPW = Q * K // NW
GCH = 4
GCW = RPW // GCH


def _topk_body(x_ref, y_ref, out_idx_ref, rb_s, rb_i, nb_s, nb_i, s_ref):
    pid = pl.program_id(1)

    @pl.when(pid == 0)
    def _init():
        rb_s[...] = jnp.full((RB, QB), jnp.inf, jnp.float32)
        rb_i[...] = jnp.full((RB, QB), BIGF, jnp.float32)

    x = x_ref[...]
    y = y_ref[...]
    y2 = jnp.sum(y * y, axis=1, keepdims=True)
    yx = lax.dot_general(y, x, (((1,), (1,)), ((), ())),
                         preferred_element_type=jnp.float32)
    row = (lax.broadcasted_iota(jnp.int32, (BLK, QB), 0).astype(jnp.float32)
           + jnp.float32(pid * BLK))
    s_ref[...] = jnp.where(row < N_BANK, y2 - 2.0 * yx, jnp.inf)

    nb_s[...] = jnp.full((RB, QB), jnp.inf, jnp.float32)
    nb_i[...] = jnp.full((RB, QB), BIGF, jnp.float32)
    for k in range(K):
        s = s_ref[...]
        m1 = jnp.min(s, axis=0, keepdims=True)
        ci_s = jnp.min(jnp.where(s == m1, row, BIGF), axis=0, keepdims=True)
        rs = rb_s[...]
        ri = rb_i[...]
        m2 = jnp.min(rs, axis=0, keepdims=True)
        use_rb = m2 <= m1
        ci_r = jnp.min(jnp.where(rs == m2, ri, BIGF), axis=0, keepdims=True)
        nb_s[pl.ds(k, 1), :] = jnp.where(use_rb, m2, m1)
        nb_i[pl.ds(k, 1), :] = jnp.where(use_rb, ci_r, ci_s)
        sel_blk = jnp.where(use_rb, -1.0, ci_s)
        sel_rb = jnp.where(use_rb, ci_r, -1.0)
        s_ref[...] = jnp.where(row == sel_blk, jnp.inf, s)
        rb_s[...] = jnp.where(ri == sel_rb, jnp.inf, rs)
    rb_s[...] = nb_s[...]
    rb_i[...] = nb_i[...]

    @pl.when(pid == NBLK - 1)
    def _emit():
        out_idx_ref[...] = nb_i[...].astype(jnp.int32)


def _mean_argmax_body(rows_ref, probs_ref, lab_ref):
    acc = rows_ref[0]
    for j in range(1, K):
        acc = acc + rows_ref[j]
    p = acc * jnp.float32(1.0 / K)
    probs_ref[...] = p
    m = jnp.max(p, axis=1, keepdims=True)
    ii = lax.broadcasted_iota(jnp.int32, (Q, C), 1)
    lab_ref[...] = jnp.min(jnp.where(p == m, ii, C), axis=1, keepdims=True)


@functools.cache
def _make_sc_gather():
    mesh = plsc.VectorSubcoreMesh(core_axis_name="c", subcore_axis_name="s")

    @functools.partial(
        pl.kernel,
        mesh=mesh,
        out_type=jax.ShapeDtypeStruct((Q * K, C), jnp.float32),
        scratch_types=[
            pltpu.VMEM((NW * GCH, GCW), jnp.int32),
            pltpu.VMEM((RPW, C), jnp.float32),
            pltpu.SemaphoreType.DMA,
        ],
        compiler_params=pltpu.CompilerParams(use_tc_tiling_on_sc=False),
    )
    def _sc_gather(idx_hbm, probs_hbm, out_hbm, idx_v, rows_v, sem):
        wid = lax.axis_index("s") * 2 + lax.axis_index("c")
        pltpu.sync_copy(idx_hbm, idx_v)
        copies = []
        for t in range(GCH):
            cp = pltpu.async_copy(
                probs_hbm.at[idx_v.at[wid * GCH + t]],
                rows_v.at[pl.ds(t * GCW, GCW)],
                sem,
            )
            copies.append(cp)
        for cp in copies:
            cp.wait()
        pltpu.sync_copy(rows_v, out_hbm.at[pl.ds(wid * RPW, RPW)])

    return _sc_gather


def _run_topk(features, features_bank):
    return pl.pallas_call(
        _topk_body,
        grid=(NQB, NBLK),
        in_specs=[
            pl.BlockSpec((QB, D), lambda q, i: (q, 0)),
            pl.BlockSpec((BLK, D), lambda q, i: (i, 0)),
        ],
        out_specs=pl.BlockSpec((RB, QB), lambda q, i: (0, q)),
        out_shape=jax.ShapeDtypeStruct((RB, Q), jnp.int32),
        scratch_shapes=[
            pltpu.VMEM((RB, QB), jnp.float32),
            pltpu.VMEM((RB, QB), jnp.float32),
            pltpu.VMEM((RB, QB), jnp.float32),
            pltpu.VMEM((RB, QB), jnp.float32),
            pltpu.VMEM((BLK, QB), jnp.float32),
        ],
    )(features, features_bank)


def _run_mean_argmax(rows3d):
    return pl.pallas_call(
        _mean_argmax_body,
        out_shape=[
            jax.ShapeDtypeStruct((Q, C), jnp.float32),
            jax.ShapeDtypeStruct((Q, 1), jnp.int32),
        ],
    )(rows3d)


def kernel(features, features_bank, probs_bank):
    top_idx = _run_topk(features, features_bank)
    idx_t = top_idx[:K].reshape(NW * GCH, GCW)
    rows = _make_sc_gather()(idx_t, probs_bank)
    probs, lab = _run_mean_argmax(rows.reshape(K, Q, C))
    return (lab[:, 0], probs)


# speedup vs baseline: 1.7339x; 1.0132x over previous
import functools

import jax
import jax.numpy as jnp
from jax import lax
from jax.experimental import pallas as pl
from jax.experimental.pallas import tpu as pltpu
from jax.experimental.pallas import tpu_sc as plsc

Q = 1024
D = 128
C = 64
K = 10
N_BANK = 100000
QB = 128
NQB = Q // QB
BLK = 512
N_PAD = 100352
NBLK = N_PAD // BLK
RB = 16
BIGI = 2**30
BIGF = 1e30
NW = 32
RPW = Q * K // NW
GCH = 4
GCW = RPW // GCH


def _topk_body(x_ref, y_ref, out_idx_ref, rb_s, rb_i, nb_s, nb_i, s_ref):
    pid = pl.program_id(0)
    qid = pl.program_id(1)

    @pl.when(pid == 0)
    def _init():
        rb_s[qid] = jnp.full((RB, QB), jnp.inf, jnp.float32)
        rb_i[qid] = jnp.full((RB, QB), BIGF, jnp.float32)

    x = x_ref[...]
    y = y_ref[...]
    y2 = jnp.sum(y * y, axis=1, keepdims=True)
    yx = lax.dot_general(y, x, (((1,), (1,)), ((), ())),
                         preferred_element_type=jnp.float32)
    row = (lax.broadcasted_iota(jnp.int32, (BLK, QB), 0).astype(jnp.float32)
           + jnp.float32(pid * BLK))
    s_ref[...] = jnp.where(row < N_BANK, y2 - 2.0 * yx, jnp.inf)

    nb_s[...] = jnp.full((RB, QB), jnp.inf, jnp.float32)
    nb_i[...] = jnp.full((RB, QB), BIGF, jnp.float32)
    for k in range(K):
        s = s_ref[...]
        m1 = jnp.min(s, axis=0, keepdims=True)
        ci_s = jnp.min(jnp.where(s == m1, row, BIGF), axis=0, keepdims=True)
        rs = rb_s[qid]
        ri = rb_i[qid]
        m2 = jnp.min(rs, axis=0, keepdims=True)
        use_rb = m2 <= m1
        ci_r = jnp.min(jnp.where(rs == m2, ri, BIGF), axis=0, keepdims=True)
        nb_s[pl.ds(k, 1), :] = jnp.where(use_rb, m2, m1)
        nb_i[pl.ds(k, 1), :] = jnp.where(use_rb, ci_r, ci_s)
        sel_blk = jnp.where(use_rb, -1.0, ci_s)
        sel_rb = jnp.where(use_rb, ci_r, -1.0)
        s_ref[...] = jnp.where(row == sel_blk, jnp.inf, s)
        rb_s[qid] = jnp.where(ri == sel_rb, jnp.inf, rs)
    rb_s[qid] = nb_s[...]
    rb_i[qid] = nb_i[...]

    @pl.when(pid == NBLK - 1)
    def _emit():
        out_idx_ref[...] = nb_i[...].astype(jnp.int32)


def _mean_argmax_body(rows_ref, probs_ref, lab_ref):
    acc = rows_ref[0]
    for j in range(1, K):
        acc = acc + rows_ref[j]
    p = acc * jnp.float32(1.0 / K)
    probs_ref[...] = p
    m = jnp.max(p, axis=1, keepdims=True)
    ii = lax.broadcasted_iota(jnp.int32, (Q, C), 1)
    lab_ref[...] = jnp.min(jnp.where(p == m, ii, C), axis=1, keepdims=True)


@functools.cache
def _make_sc_gather():
    mesh = plsc.VectorSubcoreMesh(core_axis_name="c", subcore_axis_name="s")

    @functools.partial(
        pl.kernel,
        mesh=mesh,
        out_type=jax.ShapeDtypeStruct((Q * K, C), jnp.float32),
        scratch_types=[
            pltpu.VMEM((NW * GCH, GCW), jnp.int32),
            pltpu.VMEM((RPW, C), jnp.float32),
            pltpu.SemaphoreType.DMA,
        ],
        compiler_params=pltpu.CompilerParams(use_tc_tiling_on_sc=False),
    )
    def _sc_gather(idx_hbm, probs_hbm, out_hbm, idx_v, rows_v, sem):
        wid = lax.axis_index("s") * 2 + lax.axis_index("c")
        pltpu.sync_copy(idx_hbm, idx_v)
        copies = []
        for t in range(GCH):
            cp = pltpu.async_copy(
                probs_hbm.at[idx_v.at[wid * GCH + t]],
                rows_v.at[pl.ds(t * GCW, GCW)],
                sem,
            )
            copies.append(cp)
        for cp in copies:
            cp.wait()
        pltpu.sync_copy(rows_v, out_hbm.at[pl.ds(wid * RPW, RPW)])

    return _sc_gather


def _run_topk(features, features_bank):
    return pl.pallas_call(
        _topk_body,
        grid=(NBLK, NQB),
        in_specs=[
            pl.BlockSpec((QB, D), lambda i, q: (q, 0)),
            pl.BlockSpec((BLK, D), lambda i, q: (i, 0)),
        ],
        out_specs=pl.BlockSpec((RB, QB), lambda i, q: (0, q)),
        out_shape=jax.ShapeDtypeStruct((RB, Q), jnp.int32),
        scratch_shapes=[
            pltpu.VMEM((NQB, RB, QB), jnp.float32),
            pltpu.VMEM((NQB, RB, QB), jnp.float32),
            pltpu.VMEM((RB, QB), jnp.float32),
            pltpu.VMEM((RB, QB), jnp.float32),
            pltpu.VMEM((BLK, QB), jnp.float32),
        ],
    )(features, features_bank)


def _run_mean_argmax(rows3d):
    return pl.pallas_call(
        _mean_argmax_body,
        out_shape=[
            jax.ShapeDtypeStruct((Q, C), jnp.float32),
            jax.ShapeDtypeStruct((Q, 1), jnp.int32),
        ],
    )(rows3d)


def kernel(features, features_bank, probs_bank):
    top_idx = _run_topk(features, features_bank)
    idx_t = top_idx[:K].reshape(NW * GCH, GCW)
    rows = _make_sc_gather()(idx_t, probs_bank)
    probs, lab = _run_mean_argmax(rows.reshape(K, Q, C))
    return (lab[:, 0], probs)
